# baseline (speedup 1.0000x reference)
import functools

import jax
import jax.numpy as jnp
from jax import lax
from jax.experimental import pallas as pl
from jax.experimental.pallas import tpu as pltpu
from jax.experimental.pallas import tpu_sc as plsc

N = 108000
B = 1024
L = 4096
_LP = L + 8

_info = plsc.get_sparse_core_info()
_NC = _info.num_cores
_NS = _info.num_subcores
_NW = _NC * _NS
_ROWS = B // _NW
_NG = 8
_CR = 8
_NCHUNK = _ROWS // _CR


def _make_kernel():
    mesh = plsc.VectorSubcoreMesh(core_axis_name="c", subcore_axis_name="s")

    @functools.partial(
        pl.kernel,
        out_type=jax.ShapeDtypeStruct((B * L,), jnp.float32),
        mesh=mesh,
        scratch_types=[
            pltpu.VMEM((_ROWS + 16,), jnp.int32),
            pltpu.VMEM((_NG * _LP,), jnp.float32),
            pltpu.VMEM((2, _CR * L), jnp.float32),
            pltpu.SemaphoreType.DMA,
            pltpu.SemaphoreType.DMA,
        ],
    )
    def gather_windows(data_hbm, starts_hbm, out_hbm, starts_v, inb, outb,
                       sem_g, sem_w):
        wid = lax.axis_index("s") * _NC + lax.axis_index("c")
        base = wid * _ROWS
        pltpu.sync_copy(starts_hbm.at[pl.ds(base, _ROWS)],
                        starts_v.at[pl.ds(0, _ROWS)])

        def read_start(j):
            return starts_v[pl.ds(j, 16)][0]

        def fire_gather(j):
            s0 = pl.multiple_of(read_start(j) & ~7, 8)
            pltpu.async_copy(data_hbm.at[pl.ds(s0, _LP)],
                             inb.at[pl.ds((j % _NG) * _LP, _LP)], sem_g)

        def wait_gather():
            pltpu.make_async_copy(data_hbm.at[pl.ds(0, _LP)],
                                  inb.at[pl.ds(0, _LP)], sem_g).wait()

        def wait_write():
            pltpu.make_async_copy(data_hbm.at[pl.ds(0, _CR * L)], outb.at[0],
                                  sem_w).wait()

        def shift(i, cb, k):
            s = read_start(i)
            off = (i % _NG) * _LP + (s - (s & ~7))

            @plsc.parallel_loop(0, L, step=16, unroll=16)
            def vec_body(o):
                outb[cb, pl.ds(k * L + o, 16)] = inb[pl.ds(off + o, 16)]

        for j in range(_NG):
            fire_gather(j)

        for c in range(_NCHUNK):
            cb = c & 1
            if c >= 2:
                wait_write()
            for k in range(_CR):
                i = c * _CR + k
                wait_gather()
                shift(i, cb, k)
                if i + _NG < _ROWS:
                    fire_gather(i + _NG)
            pltpu.async_copy(outb.at[cb],
                             out_hbm.at[pl.ds((base + c * _CR) * L, _CR * L)],
                             sem_w)

        wait_write()
        wait_write()

    return gather_windows


_gather = _make_kernel()


@jax.jit
def _run(data, starts):
    out = _gather(data.reshape(N), starts.astype(jnp.int32))
    return out.reshape(B, L, 1)


def kernel(data, starts, length):
    del length
    return _run(data, starts)


# speedup vs baseline: 1.3077x; 1.3077x over previous
import functools

import jax
import jax.numpy as jnp
from jax import lax
from jax.experimental import pallas as pl
from jax.experimental.pallas import tpu as pltpu
from jax.experimental.pallas import tpu_sc as plsc

N = 108000
B = 1024
L = 4096
_LP = L + 8

_info = plsc.get_sparse_core_info()
_NC = _info.num_cores
_NS = _info.num_subcores
_NW = _NC * _NS
_ROWS = B // _NW
_NB = 6


def _make_kernel():
    mesh = plsc.VectorSubcoreMesh(core_axis_name="c", subcore_axis_name="s")

    @functools.partial(
        pl.kernel,
        out_type=jax.ShapeDtypeStruct((B * L,), jnp.float32),
        mesh=mesh,
        scratch_types=[
            pltpu.VMEM_SHARED((N,), jnp.float32),
            pltpu.VMEM((_ROWS + 16,), jnp.int32),
            pltpu.VMEM((_NB * _LP,), jnp.float32),
            pltpu.VMEM((_NB, L), jnp.float32),
            pltpu.SemaphoreType.DMA,
            pltpu.SemaphoreType.DMA,
        ],
    )
    def gather_windows(data_hbm, starts_hbm, out_hbm, data_sp, starts_v, inb,
                       outb, sem_g, sem_w):
        cid = lax.axis_index("c")
        sid = lax.axis_index("s")
        wid = sid * _NC + cid
        base = wid * _ROWS
        pltpu.sync_copy(starts_hbm.at[pl.ds(base, _ROWS)],
                        starts_v.at[pl.ds(0, _ROWS)])

        @pl.when(sid == 0)
        def _stage():
            pltpu.sync_copy(data_hbm, data_sp)

        plsc.subcore_barrier()

        def read_start(j):
            return starts_v[pl.ds(j, 16)][0]

        def fire_gather(j):
            s0 = pl.multiple_of(read_start(j) & ~7, 8)
            pltpu.async_copy(data_sp.at[pl.ds(s0, _LP)],
                             inb.at[pl.ds((j % _NB) * _LP, _LP)], sem_g)

        def wait_gather():
            pltpu.make_async_copy(data_hbm.at[pl.ds(0, _LP)],
                                  inb.at[pl.ds(0, _LP)], sem_g).wait()

        def wait_write():
            pltpu.make_async_copy(data_hbm.at[pl.ds(0, L)], outb.at[0],
                                  sem_w).wait()

        def shift(i, b):
            s = read_start(i)
            off = b * _LP + (s - (s & ~7))

            @plsc.parallel_loop(0, L, step=16, unroll=16)
            def vec_body(o):
                outb[b, pl.ds(o, 16)] = inb[pl.ds(off + o, 16)]

        def fire_write(i, b):
            pltpu.async_copy(outb.at[b], out_hbm.at[pl.ds((base + i) * L, L)],
                             sem_w)

        for j in range(_NB):
            fire_gather(j)

        for i in range(_NB):
            wait_gather()
            shift(i, i % _NB)
            fire_write(i, i % _NB)
            fire_gather(i + _NB)

        def steady_body(i, carry):
            b = i % _NB
            wait_gather()
            wait_write()
            shift(i, b)
            fire_write(i, b)
            fire_gather(i + _NB)
            return carry

        lax.fori_loop(_NB, _ROWS - _NB, steady_body, 0)

        for i in range(_ROWS - _NB, _ROWS):
            wait_gather()
            wait_write()
            shift(i, i % _NB)
            fire_write(i, i % _NB)

        for _ in range(_NB):
            wait_write()

    return gather_windows


_gather = _make_kernel()


@jax.jit
def _run(data, starts):
    out = _gather(data.reshape(N), starts.astype(jnp.int32))
    return out.reshape(B, L, 1)


def kernel(data, starts, length):
    del length
    return _run(data, starts)
